# baseline (speedup 1.0000x reference)
import functools

import jax
import jax.numpy as jnp
from jax import lax
from jax.experimental import pallas as pl
from jax.experimental.pallas import tpu as pltpu
from jax.experimental.pallas import tpu_sc as plsc

NUM_ITEMS = 1000000
EMBED_DIM = 16
BATCH = 16384
CURVATURE = 1.0

_NC = 2
_NS = 16
_NW = _NC * _NS
_CHUNK = 128


def _sc_gather(parent_indices, child_indices, embeddings):
    B = parent_indices.shape[0]
    D = embeddings.shape[1]
    b_per_w = B // _NW
    n_ch = b_per_w // _CHUNK

    mesh = plsc.VectorSubcoreMesh(core_axis_name="c", subcore_axis_name="s")

    @functools.partial(
        pl.kernel,
        out_type=[
            jax.ShapeDtypeStruct((B, D), jnp.float32),
            jax.ShapeDtypeStruct((B, D), jnp.float32),
        ],
        mesh=mesh,
        compiler_params=pltpu.CompilerParams(use_tc_tiling_on_sc=False),
        scratch_types=[
            pltpu.VMEM((b_per_w,), jnp.int32),
            pltpu.VMEM((b_per_w,), jnp.int32),
            pltpu.VMEM((b_per_w, D), jnp.float32),
            pltpu.VMEM((b_per_w, D), jnp.float32),
            pltpu.SemaphoreType.DMA,
        ],
    )
    def gather_k(pidx_hbm, cidx_hbm, table_hbm, pout_hbm, cout_hbm,
                 pidx_v, cidx_v, prows_v, crows_v, sem):
        wid = lax.axis_index("s") * _NC + lax.axis_index("c")
        base = wid * b_per_w
        pltpu.sync_copy(pidx_hbm.at[pl.ds(base, b_per_w)], pidx_v)
        pltpu.sync_copy(cidx_hbm.at[pl.ds(base, b_per_w)], cidx_v)
        copies = []
        for j in range(n_ch):
            sl = pl.ds(j * _CHUNK, _CHUNK)
            copies.append(pltpu.async_copy(
                table_hbm.at[pidx_v.at[sl]], prows_v.at[sl], sem))
            copies.append(pltpu.async_copy(
                table_hbm.at[cidx_v.at[sl]], crows_v.at[sl], sem))
        for c in copies:
            c.wait()
        pltpu.sync_copy(prows_v, pout_hbm.at[pl.ds(base, b_per_w)])
        pltpu.sync_copy(crows_v, cout_hbm.at[pl.ds(base, b_per_w)])

    return gather_k(parent_indices, child_indices, embeddings)


def _tc_body(p_ref, c_ref, o_ref):
    u = p_ref[...]
    v = c_ref[...]
    eps = 1e-05
    max_norm = 1 - eps

    def project(x):
        norm = jnp.sqrt(jnp.sum(x * x, axis=-1, keepdims=True))
        scale = jnp.where(norm >= max_norm, max_norm / (norm + 1e-07), 1.0)
        return x * scale

    u = project(u)
    v = project(v)
    u_sq = jnp.sum(u * u, axis=-1)
    v_sq = jnp.sum(v * v, axis=-1)
    d_sq = jnp.sum((u - v) * (u - v), axis=-1)
    q = 2.0 * d_sq / ((1.0 - u_sq) * (1.0 - v_sq) + 1e-07)
    dist = jnp.log1p(q + jnp.sqrt(q * (q + 2.0)))
    o_ref[0, 0] = jnp.sum(dist) * (CURVATURE / p_ref.shape[0])


def _tc_distance(parent_rows, child_rows):
    out = pl.pallas_call(
        _tc_body,
        out_shape=jax.ShapeDtypeStruct((1, 1), jnp.float32),
        out_specs=pl.BlockSpec(memory_space=pltpu.SMEM),
    )(parent_rows, child_rows)
    return out[0, 0]


def kernel(parent_indices, child_indices, embeddings):
    prows, crows = _sc_gather(parent_indices, child_indices, embeddings)
    return _tc_distance(prows, crows)
